# Initial kernel scaffold; baseline (speedup 1.0000x reference)
import jax
import jax.numpy as jnp
from jax.experimental import pallas as pl


def kernel(pc1, pc2, d1, d2):
    raise NotImplementedError("write your pallas kernel here")


# speedup vs baseline: 1.0742x; 1.0742x over previous
import jax
import jax.numpy as jnp
from jax.experimental import pallas as pl

_EPS = 1e-05
_FACT = 2.0
_TN1 = 256


def _body(e1_ref, e2_ref, d1t_ref, d2_ref, pc2e_ref, out_ref):
    ip = jnp.dot(d1t_ref[...], d2_ref[...], preferred_element_type=jnp.float32)
    sqd = jnp.dot(e1_ref[...], e2_ref[...], preferred_element_type=jnp.float32)
    dist = jnp.reciprocal(jnp.maximum(sqd, _EPS)) * (ip * ip)
    s = _FACT * dist
    m = jnp.max(s, axis=1, keepdims=True)
    p = jnp.exp(s - m)
    w = p / jnp.sum(p, axis=1, keepdims=True)
    out_ref[...] = jnp.dot(w, pc2e_ref[...], preferred_element_type=jnp.float32)


def kernel(pc1, pc2, d1, d2):
    n = pc1.shape[1]
    f32 = jnp.float32
    pc1_sq = pc1 ** 2
    pc2_sq = pc2 ** 2
    e1 = jnp.ones((n, 9), dtype=pc1.dtype)
    e1 = e1.at[:, 2].set(pc1_sq[0, :]).at[:, 5].set(pc1_sq[1, :]).at[:, 8].set(pc1_sq[2, :])
    e1 = e1.at[:, 1].set(pc1[0, :]).at[:, 4].set(pc1[1, :]).at[:, 7].set(pc1[2, :])
    e2 = jnp.ones((9, n), dtype=pc2.dtype)
    e2 = e2.at[0, :].set(pc2_sq[0, :]).at[3, :].set(pc2_sq[1, :]).at[6, :].set(pc2_sq[2, :])
    e2 = e2.at[1, :].set(pc2[0, :] * -2.0).at[4, :].set(pc2[1, :] * -2.0).at[7, :].set(pc2[2, :] * -2.0)
    d1n = d1 / jnp.maximum(jnp.linalg.norm(d1, axis=0, keepdims=True), 1e-12)
    d2n = d2 / jnp.maximum(jnp.linalg.norm(d2, axis=0, keepdims=True), 1e-12)
    d1t = d1n.T
    pc2e = jnp.concatenate([pc2.T, jnp.ones((n, 1), f32)], axis=1)
    s = d2.shape[0]

    out = pl.pallas_call(
        _body,
        grid=(n // _TN1,),
        in_specs=[
            pl.BlockSpec((_TN1, 9), lambda i: (i, 0)),
            pl.BlockSpec((9, n), lambda i: (0, 0)),
            pl.BlockSpec((_TN1, s), lambda i: (i, 0)),
            pl.BlockSpec((s, n), lambda i: (0, 0)),
            pl.BlockSpec((n, 4), lambda i: (0, 0)),
        ],
        out_specs=pl.BlockSpec((_TN1, 4), lambda i: (i, 0)),
        out_shape=jax.ShapeDtypeStruct((n, 4), f32),
    )(e1, e2, d1t, d2n, pc2e)

    pc_nearest = out[:, :3].T
    indexor = jnp.ones((n,), pc1.dtype)
    return (pc_nearest, indexor)


# speedup vs baseline: 1.6990x; 1.5816x over previous
import jax
import jax.numpy as jnp
from jax.experimental import pallas as pl
from jax.experimental.pallas import tpu as pltpu

_EPS = 1e-05
_FACT = 2.0
_TN1 = 256


def _body(e1_ref, e2_ref, d1_ref, n1_ref, d2_ref, n2_ref, pc2e_ref, out_ref,
          d2n_ref):
    @pl.when(pl.program_id(0) == 0)
    def _():
        d2n_ref[...] = d2_ref[...] / n2_ref[...]

    d1n = d1_ref[...] / n1_ref[...]
    ip = jax.lax.dot_general(d1n, d2n_ref[...], (((0,), (0,)), ((), ())),
                             preferred_element_type=jnp.float32)
    sqd = jnp.dot(e1_ref[...], e2_ref[...], preferred_element_type=jnp.float32)
    dist = jnp.reciprocal(jnp.maximum(sqd, _EPS)) * (ip * ip)
    s = _FACT * dist
    m = jnp.max(s, axis=1, keepdims=True)
    p = jnp.exp(s - m)
    w = p / jnp.sum(p, axis=1, keepdims=True)
    out_ref[...] = jnp.dot(w, pc2e_ref[...], preferred_element_type=jnp.float32)


def kernel(pc1, pc2, d1, d2):
    n = pc1.shape[1]
    f32 = jnp.float32
    one = jnp.ones((n,), f32)
    x1, y1, z1 = pc1[0], pc1[1], pc1[2]
    x2, y2, z2 = pc2[0], pc2[1], pc2[2]
    e1 = jnp.stack([one, x1, x1 * x1, one, y1, y1 * y1, one, z1, z1 * z1],
                   axis=1)
    e2 = jnp.stack([x2 * x2, -2.0 * x2, one, y2 * y2, -2.0 * y2, one,
                    z2 * z2, -2.0 * z2, one], axis=0)
    pc2e = jnp.concatenate([pc2.T, jnp.ones((n, 1), f32)], axis=1)
    n1 = jnp.maximum(jnp.linalg.norm(d1, axis=0, keepdims=True), 1e-12)
    n2 = jnp.maximum(jnp.linalg.norm(d2, axis=0, keepdims=True), 1e-12)
    s = d2.shape[0]

    out = pl.pallas_call(
        _body,
        grid=(n // _TN1,),
        in_specs=[
            pl.BlockSpec((_TN1, 9), lambda i: (i, 0)),
            pl.BlockSpec((9, n), lambda i: (0, 0)),
            pl.BlockSpec((s, _TN1), lambda i: (0, i)),
            pl.BlockSpec((1, _TN1), lambda i: (0, i)),
            pl.BlockSpec((s, n), lambda i: (0, 0)),
            pl.BlockSpec((1, n), lambda i: (0, 0)),
            pl.BlockSpec((n, 4), lambda i: (0, 0)),
        ],
        out_specs=pl.BlockSpec((_TN1, 4), lambda i: (i, 0)),
        out_shape=jax.ShapeDtypeStruct((n, 4), f32),
        scratch_shapes=[pltpu.VMEM((s, n), f32)],
    )(e1, e2, d1, n1, d2, n2, pc2e)

    pc_nearest = out[:, :3].T
    indexor = jnp.ones((n,), pc1.dtype)
    return (pc_nearest, indexor)


# speedup vs baseline: 1.7504x; 1.0303x over previous
import jax
import jax.numpy as jnp
from jax.experimental import pallas as pl
from jax.experimental.pallas import tpu as pltpu

_EPS = 1e-05
_FACT = 2.0
_TN1 = 512


def _body(e1_ref, e2_ref, d1_ref, n1_ref, d2_ref, n2_ref, pc2e_ref, out_ref,
          d2n_ref):
    @pl.when(pl.program_id(0) == 0)
    def _():
        d2n_ref[...] = d2_ref[...] / n2_ref[...]

    d1n = d1_ref[...] / n1_ref[...]
    ip = jax.lax.dot_general(d1n, d2n_ref[...], (((0,), (0,)), ((), ())),
                             preferred_element_type=jnp.float32)
    sqd = jnp.dot(e1_ref[...], e2_ref[...], preferred_element_type=jnp.float32)
    dist = jnp.reciprocal(jnp.maximum(sqd, _EPS)) * (ip * ip)
    s = _FACT * dist
    m = jnp.max(s, axis=1, keepdims=True)
    p = jnp.exp(s - m)
    w = p / jnp.sum(p, axis=1, keepdims=True)
    out_ref[...] = jnp.dot(w, pc2e_ref[...], preferred_element_type=jnp.float32)


def kernel(pc1, pc2, d1, d2):
    n = pc1.shape[1]
    f32 = jnp.float32
    one = jnp.ones((n,), f32)
    x1, y1, z1 = pc1[0], pc1[1], pc1[2]
    x2, y2, z2 = pc2[0], pc2[1], pc2[2]
    e1 = jnp.stack([one, x1, x1 * x1, one, y1, y1 * y1, one, z1, z1 * z1],
                   axis=1)
    e2 = jnp.stack([x2 * x2, -2.0 * x2, one, y2 * y2, -2.0 * y2, one,
                    z2 * z2, -2.0 * z2, one], axis=0)
    pc2e = jnp.concatenate([pc2.T, jnp.ones((n, 1), f32)], axis=1)
    n1 = jnp.maximum(jnp.linalg.norm(d1, axis=0, keepdims=True), 1e-12)
    n2 = jnp.maximum(jnp.linalg.norm(d2, axis=0, keepdims=True), 1e-12)
    s = d2.shape[0]

    out = pl.pallas_call(
        _body,
        grid=(n // _TN1,),
        in_specs=[
            pl.BlockSpec((_TN1, 9), lambda i: (i, 0)),
            pl.BlockSpec((9, n), lambda i: (0, 0)),
            pl.BlockSpec((s, _TN1), lambda i: (0, i)),
            pl.BlockSpec((1, _TN1), lambda i: (0, i)),
            pl.BlockSpec((s, n), lambda i: (0, 0)),
            pl.BlockSpec((1, n), lambda i: (0, 0)),
            pl.BlockSpec((n, 4), lambda i: (0, 0)),
        ],
        out_specs=pl.BlockSpec((_TN1, 4), lambda i: (i, 0)),
        out_shape=jax.ShapeDtypeStruct((n, 4), f32),
        scratch_shapes=[pltpu.VMEM((s, n), f32)],
    )(e1, e2, d1, n1, d2, n2, pc2e)

    pc_nearest = out[:, :3].T
    indexor = jnp.ones((n,), pc1.dtype)
    return (pc_nearest, indexor)


# speedup vs baseline: 1.8008x; 1.0288x over previous
import jax
import jax.numpy as jnp
from jax.experimental import pallas as pl
from jax.experimental.pallas import tpu as pltpu

_EPS = 1e-05
_FACT = 2.0
_TN1 = 512


def _body(e1_ref, e2_ref, d1_ref, n1_ref, d2_ref, n2_ref, pc2e_ref, out_ref,
          d2n_ref):
    @pl.when(pl.program_id(0) == 0)
    def _():
        d2n_ref[...] = d2_ref[...] / n2_ref[...]

    d1n = d1_ref[...] / n1_ref[...]
    ip = jax.lax.dot_general(d1n, d2n_ref[...], (((0,), (0,)), ((), ())),
                             preferred_element_type=jnp.float32)
    sqd = jnp.dot(e1_ref[...], e2_ref[...], preferred_element_type=jnp.float32)
    dist = jnp.reciprocal(jnp.maximum(sqd, _EPS)) * (ip * ip)
    m = jnp.max(dist, axis=1, keepdims=True)
    p = jnp.exp2((dist - m) * jnp.float32(_FACT * 1.4426950408889634))
    w = p / jnp.sum(p, axis=1, keepdims=True)
    out_ref[...] = jnp.dot(w, pc2e_ref[...], preferred_element_type=jnp.float32)


def kernel(pc1, pc2, d1, d2):
    n = pc1.shape[1]
    f32 = jnp.float32
    one = jnp.ones((n,), f32)
    x1, y1, z1 = pc1[0], pc1[1], pc1[2]
    x2, y2, z2 = pc2[0], pc2[1], pc2[2]
    e1 = jnp.stack([one, x1, x1 * x1, one, y1, y1 * y1, one, z1, z1 * z1],
                   axis=1)
    e2 = jnp.stack([x2 * x2, -2.0 * x2, one, y2 * y2, -2.0 * y2, one,
                    z2 * z2, -2.0 * z2, one], axis=0)
    pc2e = jnp.concatenate([pc2.T, jnp.ones((n, 1), f32)], axis=1)
    n1 = jnp.maximum(jnp.linalg.norm(d1, axis=0, keepdims=True), 1e-12)
    n2 = jnp.maximum(jnp.linalg.norm(d2, axis=0, keepdims=True), 1e-12)
    s = d2.shape[0]

    out = pl.pallas_call(
        _body,
        grid=(n // _TN1,),
        in_specs=[
            pl.BlockSpec((_TN1, 9), lambda i: (i, 0)),
            pl.BlockSpec((9, n), lambda i: (0, 0)),
            pl.BlockSpec((s, _TN1), lambda i: (0, i)),
            pl.BlockSpec((1, _TN1), lambda i: (0, i)),
            pl.BlockSpec((s, n), lambda i: (0, 0)),
            pl.BlockSpec((1, n), lambda i: (0, 0)),
            pl.BlockSpec((n, 4), lambda i: (0, 0)),
        ],
        out_specs=pl.BlockSpec((_TN1, 4), lambda i: (i, 0)),
        out_shape=jax.ShapeDtypeStruct((n, 4), f32),
        scratch_shapes=[pltpu.VMEM((s, n), f32)],
    )(e1, e2, d1, n1, d2, n2, pc2e)

    pc_nearest = out[:, :3].T
    indexor = jnp.ones((n,), pc1.dtype)
    return (pc_nearest, indexor)


# speedup vs baseline: 1.9307x; 1.0721x over previous
import jax
import jax.numpy as jnp
from jax.experimental import pallas as pl
from jax.experimental.pallas import tpu as pltpu

_EPS = 1e-05
_FACT = 2.0
_TN1 = 512


def _body(e1_ref, e2_ref, d1_ref, n1_ref, d2_ref, n2_ref, pc2e_ref, out_ref,
          d2n_ref):
    @pl.when(pl.program_id(0) == 0)
    def _():
        d2n_ref[...] = d2_ref[...] / n2_ref[...]

    halves = []
    h = _TN1 // 2
    for k in range(2):
        sl = pl.ds(k * h, h)
        d1n = d1_ref[:, sl] / n1_ref[:, sl]
        ip = jax.lax.dot_general(d1n, d2n_ref[...], (((0,), (0,)), ((), ())),
                                 preferred_element_type=jnp.float32)
        sqd = jnp.dot(e1_ref[sl, :], e2_ref[...],
                      preferred_element_type=jnp.float32)
        dist = jnp.reciprocal(jnp.maximum(sqd, _EPS)) * (ip * ip)
        m = jnp.max(dist, axis=1, keepdims=True)
        p = jnp.exp2((dist - m) * jnp.float32(_FACT * 1.4426950408889634))
        w = p / jnp.sum(p, axis=1, keepdims=True)
        halves.append(jnp.dot(w, pc2e_ref[...],
                              preferred_element_type=jnp.float32))
    out_ref[...] = jnp.concatenate(halves, axis=0)


def kernel(pc1, pc2, d1, d2):
    n = pc1.shape[1]
    f32 = jnp.float32
    one = jnp.ones((n,), f32)
    x1, y1, z1 = pc1[0], pc1[1], pc1[2]
    x2, y2, z2 = pc2[0], pc2[1], pc2[2]
    e1 = jnp.stack([one, x1, x1 * x1, one, y1, y1 * y1, one, z1, z1 * z1],
                   axis=1)
    e2 = jnp.stack([x2 * x2, -2.0 * x2, one, y2 * y2, -2.0 * y2, one,
                    z2 * z2, -2.0 * z2, one], axis=0)
    pc2e = jnp.concatenate([pc2.T, jnp.ones((n, 1), f32)], axis=1)
    n1 = jnp.maximum(jnp.linalg.norm(d1, axis=0, keepdims=True), 1e-12)
    n2 = jnp.maximum(jnp.linalg.norm(d2, axis=0, keepdims=True), 1e-12)
    s = d2.shape[0]

    out = pl.pallas_call(
        _body,
        grid=(n // _TN1,),
        in_specs=[
            pl.BlockSpec((_TN1, 9), lambda i: (i, 0)),
            pl.BlockSpec((9, n), lambda i: (0, 0)),
            pl.BlockSpec((s, _TN1), lambda i: (0, i)),
            pl.BlockSpec((1, _TN1), lambda i: (0, i)),
            pl.BlockSpec((s, n), lambda i: (0, 0)),
            pl.BlockSpec((1, n), lambda i: (0, 0)),
            pl.BlockSpec((n, 4), lambda i: (0, 0)),
        ],
        out_specs=pl.BlockSpec((_TN1, 4), lambda i: (i, 0)),
        out_shape=jax.ShapeDtypeStruct((n, 4), f32),
        scratch_shapes=[pltpu.VMEM((s, n), f32)],
    )(e1, e2, d1, n1, d2, n2, pc2e)

    pc_nearest = out[:, :3].T
    indexor = jnp.ones((n,), pc1.dtype)
    return (pc_nearest, indexor)


# speedup vs baseline: 2.1121x; 1.0939x over previous
import jax
import jax.numpy as jnp
from jax.experimental import pallas as pl
from jax.experimental.pallas import tpu as pltpu

_EPS = 1e-05
_FACT = 2.0
_TN1 = 512


def _body(e1_ref, e2_ref, d1_ref, n1_ref, d2_ref, n2_ref, pc2e_ref, out_ref,
          d2n_ref):
    @pl.when(pl.program_id(0) == 0)
    def _():
        d2n_ref[...] = d2_ref[...] / n2_ref[...]

    halves = []
    h = _TN1 // 4
    for k in range(4):
        sl = pl.ds(k * h, h)
        d1n = d1_ref[:, sl] / n1_ref[:, sl]
        ip = jax.lax.dot_general(d1n, d2n_ref[...], (((0,), (0,)), ((), ())),
                                 preferred_element_type=jnp.float32)
        sqd = jnp.dot(e1_ref[sl, :], e2_ref[...],
                      preferred_element_type=jnp.float32)
        dist = jnp.reciprocal(jnp.maximum(sqd, _EPS)) * (ip * ip)
        m = jnp.max(dist, axis=1, keepdims=True)
        p = jnp.exp2((dist - m) * jnp.float32(_FACT * 1.4426950408889634))
        w = p / jnp.sum(p, axis=1, keepdims=True)
        halves.append(jnp.dot(w, pc2e_ref[...],
                              preferred_element_type=jnp.float32))
    out_ref[...] = jnp.concatenate(halves, axis=0)


def kernel(pc1, pc2, d1, d2):
    n = pc1.shape[1]
    f32 = jnp.float32
    one = jnp.ones((n,), f32)
    x1, y1, z1 = pc1[0], pc1[1], pc1[2]
    x2, y2, z2 = pc2[0], pc2[1], pc2[2]
    e1 = jnp.stack([one, x1, x1 * x1, one, y1, y1 * y1, one, z1, z1 * z1],
                   axis=1)
    e2 = jnp.stack([x2 * x2, -2.0 * x2, one, y2 * y2, -2.0 * y2, one,
                    z2 * z2, -2.0 * z2, one], axis=0)
    pc2e = jnp.concatenate([pc2.T, jnp.ones((n, 1), f32)], axis=1)
    n1 = jnp.maximum(jnp.linalg.norm(d1, axis=0, keepdims=True), 1e-12)
    n2 = jnp.maximum(jnp.linalg.norm(d2, axis=0, keepdims=True), 1e-12)
    s = d2.shape[0]

    out = pl.pallas_call(
        _body,
        grid=(n // _TN1,),
        in_specs=[
            pl.BlockSpec((_TN1, 9), lambda i: (i, 0)),
            pl.BlockSpec((9, n), lambda i: (0, 0)),
            pl.BlockSpec((s, _TN1), lambda i: (0, i)),
            pl.BlockSpec((1, _TN1), lambda i: (0, i)),
            pl.BlockSpec((s, n), lambda i: (0, 0)),
            pl.BlockSpec((1, n), lambda i: (0, 0)),
            pl.BlockSpec((n, 4), lambda i: (0, 0)),
        ],
        out_specs=pl.BlockSpec((_TN1, 4), lambda i: (i, 0)),
        out_shape=jax.ShapeDtypeStruct((n, 4), f32),
        scratch_shapes=[pltpu.VMEM((s, n), f32)],
    )(e1, e2, d1, n1, d2, n2, pc2e)

    pc_nearest = out[:, :3].T
    indexor = jnp.ones((n,), pc1.dtype)
    return (pc_nearest, indexor)
